# baseline (speedup 1.0000x reference)
import dataclasses
import functools

import jax
import jax.numpy as jnp
from jax import lax
from jax.experimental import pallas as pl
from jax.experimental.pallas import tpu as pltpu
from jax.experimental.pallas import tpu_sc as plsc

_NC = 2
_NS = 16
_NW = _NC * _NS
_L = 16


def _mf_kernel(B, F, u_emb, i_emb, u_bias, i_bias, u_idx, i_idx):
    b_per_w = B // _NW
    mesh = plsc.VectorSubcoreMesh(core_axis_name="c", subcore_axis_name="s")
    cp = pltpu.CompilerParams(
        needs_layout_passes=False,
        use_tc_tiling_on_sc=False,
    )

    @functools.partial(
        pl.kernel,
        mesh=mesh,
        compiler_params=cp,
        out_type=jax.ShapeDtypeStruct((B,), jnp.float32),
        scratch_types=[
            pltpu.VMEM((b_per_w,), jnp.int32),
            pltpu.VMEM((b_per_w,), jnp.int32),
            pltpu.VMEM((b_per_w, F), jnp.float32),
            pltpu.VMEM((b_per_w, F), jnp.float32),
            pltpu.VMEM((b_per_w,), jnp.float32),
            pltpu.VMEM((b_per_w,), jnp.float32),
            pltpu.VMEM((b_per_w,), jnp.float32),
            pltpu.SemaphoreType.DMA,
        ],
    )
    def k(u_emb_hbm, i_emb_hbm, u_bias_hbm, i_bias_hbm, u_idx_hbm, i_idx_hbm,
          out_hbm, uidx_v, iidx_v, urows_v, irows_v, ub_v, ib_v, out_v, sem):
        wid = lax.axis_index("s") * _NC + lax.axis_index("c")
        base = wid * b_per_w

        pltpu.sync_copy(u_idx_hbm.at[pl.ds(base, b_per_w)], uidx_v)
        pltpu.sync_copy(i_idx_hbm.at[pl.ds(base, b_per_w)], iidx_v)

        c0 = pltpu.async_copy(u_emb_hbm.at[uidx_v], urows_v, sem)
        c1 = pltpu.async_copy(i_emb_hbm.at[iidx_v], irows_v, sem)
        c2 = pltpu.async_copy(u_bias_hbm.at[uidx_v], ub_v, sem)
        c3 = pltpu.async_copy(i_bias_hbm.at[iidx_v], ib_v, sem)
        c0.wait()
        c1.wait()
        c2.wait()
        c3.wait()

        lane = lax.iota(jnp.int32, _L)

        @pl.loop(0, b_per_w // _L)
        def _(g):
            rb = g * _L
            out16 = ub_v[pl.ds(rb, _L)] + ib_v[pl.ds(rb, _L)]
            for w in range(_L):
                acc = (urows_v[rb + w, pl.ds(0, _L)]
                       * irows_v[rb + w, pl.ds(0, _L)])
                for fb in range(1, F // _L):
                    acc = acc + (urows_v[rb + w, pl.ds(fb * _L, _L)]
                                 * irows_v[rb + w, pl.ds(fb * _L, _L)])
                s = jnp.sum(acc)
                out16 = out16 + jnp.where(lane == w, s, 0.0)
            out_v[pl.ds(rb, _L)] = out16

        pltpu.sync_copy(out_v, out_hbm.at[pl.ds(base, b_per_w)])

    return k(u_emb, i_emb, u_bias, i_bias, u_idx, i_idx)


@jax.jit
def kernel(u_emb, i_emb, u_bias, i_bias, u_idx, i_idx):
    B = u_idx.shape[0]
    F = u_emb.shape[1]
    return _mf_kernel(
        B, F, u_emb, i_emb,
        u_bias.reshape(-1), i_bias.reshape(-1),
        u_idx.astype(jnp.int32), i_idx.astype(jnp.int32),
    )
